# Initial kernel scaffold; baseline (speedup 1.0000x reference)
import jax
import jax.numpy as jnp
from jax.experimental import pallas as pl


def kernel(x_in, x_s, feature_msb, feature_lsb):
    raise NotImplementedError("write your pallas kernel here")


# speedup vs baseline: 17.9898x; 17.9898x over previous
import functools

import jax
import jax.numpy as jnp
from jax import lax
from jax.experimental import pallas as pl
from jax.experimental.pallas import tpu as pltpu, tpu_sc as plsc

_NUM_ROWS = 65536
_F = 32
_NPIX = 512 * 512
_NC = 2
_NS = 16
_NW = _NC * _NS
_CHUNK = _NPIX // _NW
_VECS = _CHUNK // 16
_BINS = 4096
_BPC = _BINS // _NS

_mesh = plsc.VectorSubcoreMesh(core_axis_name="c", subcore_axis_name="s")


@functools.partial(
    pl.kernel,
    out_type=jax.ShapeDtypeStruct((_NC * _F,), jnp.float32),
    mesh=_mesh,
    compiler_params=pltpu.CompilerParams(needs_layout_passes=False),
    scratch_types=[
        pltpu.VMEM((_CHUNK,), jnp.float32),
        pltpu.VMEM((_CHUNK,), jnp.float32),
        pltpu.VMEM((_CHUNK,), jnp.float32),
        pltpu.VMEM((_CHUNK,), jnp.float32),
        pltpu.VMEM((_CHUNK,), jnp.float32),
        pltpu.VMEM((_CHUNK,), jnp.float32),
        pltpu.VMEM((_BINS,), jnp.float32),
        pltpu.VMEM((_BINS,), jnp.float32),
        pltpu.VMEM((_BPC,), jnp.float32),
        pltpu.VMEM((_BPC,), jnp.float32),
        pltpu.VMEM((_BPC,), jnp.float32),
        pltpu.VMEM((2, 128), jnp.int32),
        pltpu.VMEM((128, 128), jnp.float32),
        pltpu.VMEM((128, 128), jnp.float32),
        pltpu.VMEM((_F,), jnp.float32),
        pltpu.VMEM((_NS, _F), jnp.float32),
        pltpu.VMEM_SHARED((_NS, _BINS), jnp.float32),
        pltpu.VMEM_SHARED((_NS, _BINS), jnp.float32),
        pltpu.VMEM_SHARED((_NS, _F), jnp.float32),
        pltpu.SemaphoreType.DMA,
    ],
)
def _featlut_sc(xa_hbm, xs_hbm, tm_hbm, tl_hbm, out_hbm,
                xa0, xa1, xa2, xs0, xs1, xs2, hm, hl, tmp, hmg, hlg,
                gidx, rows_m, rows_l, part, pv, Hm, Hl, Parts, sem):
    c = lax.axis_index("c")
    s = lax.axis_index("s")
    wid = s * _NC + c
    base = wid * _CHUNK

    copies = []
    for buf, off in ((xa0, 0), (xa1, _NPIX), (xa2, 2 * _NPIX)):
        copies.append(pltpu.async_copy(xa_hbm.at[pl.ds(off + base, _CHUNK)],
                                       buf, sem))
    for buf, off in ((xs0, 0), (xs1, _NPIX), (xs2, 2 * _NPIX)):
        copies.append(pltpu.async_copy(xs_hbm.at[pl.ds(off + base, _CHUNK)],
                                       buf, sem))

    zf = jnp.zeros((16,), jnp.float32)

    def zero_body(i, _):
        hm[pl.ds(i * 16, 16)] = zf
        hl[pl.ds(i * 16, 16)] = zf
        return 0

    lax.fori_loop(0, _BINS // 16, zero_body, 0)
    for cp in copies:
        cp.wait()

    ones = jnp.ones((16,), jnp.float32)

    def hist_body(i, _):
        off = i * 16
        a = xa0[pl.ds(off, 16)]
        b = xa1[pl.ds(off, 16)]
        d = xa2[pl.ds(off, 16)]
        im = (a * 256.0 + b * 16.0 + d).astype(jnp.int32)
        plsc.addupdate_scatter(hm, [im], ones)
        a = xs0[pl.ds(off, 16)]
        b = xs1[pl.ds(off, 16)]
        d = xs2[pl.ds(off, 16)]
        il = (a * 256.0 + b * 16.0 + d).astype(jnp.int32)
        plsc.addupdate_scatter(hl, [il], ones)
        return 0

    lax.fori_loop(0, _VECS, hist_body, 0)

    pltpu.sync_copy(hm, Hm.at[s])
    pltpu.sync_copy(hl, Hl.at[s])
    plsc.subcore_barrier()

    bin_base = s * _BPC
    for j2 in range(2):
        for i in range(8):
            row0 = (bin_base + j2 * 128 + i * 16) * 4
            gidx[j2, pl.ds(i * 16, 16)] = (
                lax.iota(jnp.int32, 16) * 4 + row0)
    gathers = [
        pltpu.async_copy(tm_hbm.at[gidx.at[0]], rows_m, sem),
        pltpu.async_copy(tl_hbm.at[gidx.at[0]], rows_l, sem),
    ]

    pltpu.sync_copy(Hm.at[0, pl.ds(bin_base, _BPC)], hmg)
    pltpu.sync_copy(Hl.at[0, pl.ds(bin_base, _BPC)], hlg)

    def merge_body(r, _):
        pltpu.sync_copy(Hm.at[r, pl.ds(bin_base, _BPC)], tmp)
        for j in range(_BPC // 16):
            hmg[pl.ds(j * 16, 16)] += tmp[pl.ds(j * 16, 16)]
        pltpu.sync_copy(Hl.at[r, pl.ds(bin_base, _BPC)], tmp)
        for j in range(_BPC // 16):
            hlg[pl.ds(j * 16, 16)] += tmp[pl.ds(j * 16, 16)]
        return 0

    lax.fori_loop(1, _NS, merge_body, 0)

    zeros = jnp.zeros((16,), jnp.float32)
    lo = zeros
    hi = zeros
    for j2 in range(2):
        for g in gathers:
            g.wait()

        def wsum_body(jv, carry, j2=j2):
            lo, hi = carry
            hmv = hmg[pl.ds(j2 * 128 + jv * 16, 16)]
            hlv = hlg[pl.ds(j2 * 128 + jv * 16, 16)]
            for k in range(16):
                j = jv * 16 + k
                rm = rows_m.at[j]
                rl = rows_l.at[j]
                lo = (lo + hmv[k] * rm[pl.ds(0, 16)]
                      + hlv[k] * rl[pl.ds(0, 16)])
                hi = (hi + hmv[k] * rm[pl.ds(16, 16)]
                      + hlv[k] * rl[pl.ds(16, 16)])
            return lo, hi

        lo, hi = lax.fori_loop(0, 8, wsum_body, (lo, hi))
        if j2 == 0:
            gathers = [
                pltpu.async_copy(tm_hbm.at[gidx.at[1]], rows_m, sem),
                pltpu.async_copy(tl_hbm.at[gidx.at[1]], rows_l, sem),
            ]

    part[pl.ds(0, 16)] = lo
    part[pl.ds(16, 16)] = hi
    pltpu.sync_copy(part, Parts.at[s])
    plsc.subcore_barrier()

    @pl.when(s == 0)
    def _():
        pltpu.sync_copy(Parts, pv)

        def red_body(r, carry):
            alo, ahi = carry
            row = pv.at[r]
            return alo + row[pl.ds(0, 16)], ahi + row[pl.ds(16, 16)]

        alo, ahi = lax.fori_loop(0, _NS, red_body, (zeros, zeros))
        part[pl.ds(0, 16)] = alo
        part[pl.ds(16, 16)] = ahi
        pltpu.sync_copy(part, out_hbm.at[pl.ds(c * _F, _F)])


def kernel(x_in, x_s, feature_msb, feature_lsb):
    xa = x_in.reshape(3 * _NPIX)
    xs = x_s.reshape(3 * _NPIX)
    tm = feature_msb.reshape(_NUM_ROWS * _F // 128, 128)
    tl = feature_lsb.reshape(_NUM_ROWS * _F // 128, 128)
    parts = _featlut_sc(xa, xs, tm, tl).reshape(_NC, _F)
    out = (parts[0] + parts[1]) * (1.0 / _NPIX)
    out = jnp.clip(jnp.round(out * 2.0) / 2.0, -16.0, 15.5)
    return out.reshape(1, _F, 1, 1)


# speedup vs baseline: 28.9554x; 1.6095x over previous
import functools

import jax
import jax.numpy as jnp
from jax import lax
from jax.experimental import pallas as pl
from jax.experimental.pallas import tpu as pltpu, tpu_sc as plsc

_NUM_ROWS = 65536
_F = 32
_NPIX = 512 * 512
_NC = 2
_NS = 16
_NW = _NC * _NS
_BINS = 4096

_mesh = plsc.VectorSubcoreMesh(core_axis_name="c", subcore_axis_name="s")


@functools.partial(
    pl.kernel,
    out_type=(
        jax.ShapeDtypeStruct((_NW, _BINS), jnp.float32),
        jax.ShapeDtypeStruct((_NW, _BINS), jnp.float32),
    ),
    mesh=_mesh,
    compiler_params=pltpu.CompilerParams(needs_layout_passes=False),
    scratch_types=[
        pltpu.VMEM((16, 512), jnp.float32),
        pltpu.VMEM((16, 512), jnp.float32),
        pltpu.VMEM((16, 512), jnp.float32),
        pltpu.VMEM((16, 512), jnp.float32),
        pltpu.VMEM((16, 512), jnp.float32),
        pltpu.VMEM((16, 512), jnp.float32),
        pltpu.VMEM((_BINS,), jnp.float32),
        pltpu.VMEM((_BINS,), jnp.float32),
        pltpu.SemaphoreType.DMA,
    ],
)
def _hist_sc(xa_hbm, xs_hbm, hm_out, hl_out,
             xa0, xa1, xa2, xs0, xs1, xs2, hm, hl, sem):
    c = lax.axis_index("c")
    s = lax.axis_index("s")
    wid = s * _NC + c
    row_base = wid * 16

    copies = []
    for ch, buf in enumerate((xa0, xa1, xa2)):
        copies.append(pltpu.async_copy(
            xa_hbm.at[0, ch, pl.ds(row_base, 16), :], buf, sem))
    for ch, buf in enumerate((xs0, xs1, xs2)):
        copies.append(pltpu.async_copy(
            xs_hbm.at[0, ch, pl.ds(row_base, 16), :], buf, sem))

    zf = jnp.zeros((16,), jnp.float32)

    def zero_body(i, _):
        hm[pl.ds(i * 16, 16)] = zf
        hl[pl.ds(i * 16, 16)] = zf
        return 0

    lax.fori_loop(0, _BINS // 16, zero_body, 0)
    for cp in copies:
        cp.wait()

    ones = jnp.ones((16,), jnp.float32)

    def hist_body(r, _):
        for j in range(512 // 16):
            sl = pl.ds(j * 16, 16)
            a = xa0[r, sl]
            b = xa1[r, sl]
            d = xa2[r, sl]
            im = (a * 256.0 + b * 16.0 + d).astype(jnp.int32)
            plsc.addupdate_scatter(hm, [im], ones)
            a = xs0[r, sl]
            b = xs1[r, sl]
            d = xs2[r, sl]
            il = (a * 256.0 + b * 16.0 + d).astype(jnp.int32)
            plsc.addupdate_scatter(hl, [il], ones)
        return 0

    lax.fori_loop(0, 16, hist_body, 0)

    pltpu.sync_copy(hm, hm_out.at[wid])
    pltpu.sync_copy(hl, hl_out.at[wid])


def _compact_body(tm_ref, tl_ref, om_ref, ol_ref):
    om_ref[...] = tm_ref[::16, :]
    ol_ref[...] = tl_ref[::16, :]


_compact = pl.pallas_call(
    _compact_body,
    grid=(16,),
    in_specs=[
        pl.BlockSpec((_BINS, _F), lambda i: (i, 0)),
        pl.BlockSpec((_BINS, _F), lambda i: (i, 0)),
    ],
    out_specs=[
        pl.BlockSpec((_BINS // 16, _F), lambda i: (i, 0)),
        pl.BlockSpec((_BINS // 16, _F), lambda i: (i, 0)),
    ],
    out_shape=[
        jax.ShapeDtypeStruct((_BINS, _F), jnp.float32),
        jax.ShapeDtypeStruct((_BINS, _F), jnp.float32),
    ],
)


def _finish_body(hm_ref, hl_ref, t16m_ref, t16l_ref, o_ref):
    hm = jnp.sum(hm_ref[...], axis=0, keepdims=True)
    hl = jnp.sum(hl_ref[...], axis=0, keepdims=True)
    acc = jnp.dot(hm, t16m_ref[...], preferred_element_type=jnp.float32)
    acc = acc + jnp.dot(hl, t16l_ref[...], preferred_element_type=jnp.float32)
    o = acc * (1.0 / _NPIX)
    o = jnp.clip(jnp.round(o * 2.0) * 0.5, -16.0, 15.5)
    o_ref[...] = o


_finish = pl.pallas_call(
    _finish_body,
    out_shape=jax.ShapeDtypeStruct((1, _F), jnp.float32),
)


def kernel(x_in, x_s, feature_msb, feature_lsb):
    tm = feature_msb.reshape(_NUM_ROWS, _F)
    tl = feature_lsb.reshape(_NUM_ROWS, _F)
    hm32, hl32 = _hist_sc(x_in, x_s)
    t16m, t16l = _compact(tm, tl)
    out = _finish(hm32, hl32, t16m, t16l)
    return out.reshape(1, _F, 1, 1)


# speedup vs baseline: 57.9917x; 2.0028x over previous
import functools

import jax
import jax.numpy as jnp
from jax import lax
from jax.experimental import pallas as pl
from jax.experimental.pallas import tpu as pltpu, tpu_sc as plsc

_NUM_ROWS = 65536
_F = 32
_NPIX = 512 * 512
_NC = 2
_NS = 16
_NW = _NC * _NS
_BINS = 4096

_mesh = plsc.VectorSubcoreMesh(core_axis_name="c", subcore_axis_name="s")


@functools.partial(
    pl.kernel,
    out_type=(
        jax.ShapeDtypeStruct((_NW, _BINS), jnp.float32),
        jax.ShapeDtypeStruct((_NW, _BINS), jnp.float32),
    ),
    mesh=_mesh,
    compiler_params=pltpu.CompilerParams(needs_layout_passes=False),
    scratch_types=[
        pltpu.VMEM((16, 512), jnp.float32),
        pltpu.VMEM((16, 512), jnp.float32),
        pltpu.VMEM((16, 512), jnp.float32),
        pltpu.VMEM((16, 512), jnp.float32),
        pltpu.VMEM((16, 512), jnp.float32),
        pltpu.VMEM((16, 512), jnp.float32),
        pltpu.VMEM((_BINS,), jnp.float32),
        pltpu.VMEM((_BINS,), jnp.float32),
        pltpu.SemaphoreType.DMA,
    ],
)
def _hist_sc(xa_hbm, xs_hbm, hm_out, hl_out,
             xa0, xa1, xa2, xs0, xs1, xs2, hm, hl, sem):
    c = lax.axis_index("c")
    s = lax.axis_index("s")
    wid = s * _NC + c
    row_base = wid * 16

    copies = []
    for ch, buf in enumerate((xa0, xa1, xa2)):
        copies.append(pltpu.async_copy(
            xa_hbm.at[0, ch, pl.ds(row_base, 16), :], buf, sem))
    for ch, buf in enumerate((xs0, xs1, xs2)):
        copies.append(pltpu.async_copy(
            xs_hbm.at[0, ch, pl.ds(row_base, 16), :], buf, sem))

    zf = jnp.zeros((16,), jnp.float32)

    def zero_body(i, _):
        hm[pl.ds(i * 16, 16)] = zf
        hl[pl.ds(i * 16, 16)] = zf
        return 0

    lax.fori_loop(0, _BINS // 16, zero_body, 0)
    for cp in copies:
        cp.wait()

    ones = jnp.ones((16,), jnp.float32)

    def hist_body(i, _):
        r = i >> 1
        hj = (i & 1) * 16
        for g in range(8):
            s0 = pl.ds((hj + 2 * g) * 16, 16)
            s1 = pl.ds((hj + 2 * g + 1) * 16, 16)
            a0 = xa0[r, s0]
            b0 = xa1[r, s0]
            d0 = xa2[r, s0]
            a1 = xs0[r, s0]
            b1 = xs1[r, s0]
            d1 = xs2[r, s0]
            a2 = xa0[r, s1]
            b2 = xa1[r, s1]
            d2 = xa2[r, s1]
            a3 = xs0[r, s1]
            b3 = xs1[r, s1]
            d3 = xs2[r, s1]
            i0 = (a0 * 256.0 + b0 * 16.0 + d0).astype(jnp.int32)
            i1 = (a1 * 256.0 + b1 * 16.0 + d1).astype(jnp.int32)
            i2 = (a2 * 256.0 + b2 * 16.0 + d2).astype(jnp.int32)
            i3 = (a3 * 256.0 + b3 * 16.0 + d3).astype(jnp.int32)
            i0 = ((i0 & 7) << 9) | (i0 >> 3)
            i1 = ((i1 & 7) << 9) | (i1 >> 3)
            i2 = ((i2 & 7) << 9) | (i2 >> 3)
            i3 = ((i3 & 7) << 9) | (i3 >> 3)
            plsc.addupdate_scatter(hm, [i0], ones)
            plsc.addupdate_scatter(hl, [i1], ones)
            plsc.addupdate_scatter(hm, [i2], ones)
            plsc.addupdate_scatter(hl, [i3], ones)
        return 0

    lax.fori_loop(0, 32, hist_body, 0)

    pltpu.sync_copy(hm, hm_out.at[wid])
    pltpu.sync_copy(hl, hl_out.at[wid])


_WROWS = _NUM_ROWS * _F // 128
_BROWS = _WROWS // 4


def _compact_body(tm_ref, tl_ref, om_ref, ol_ref):
    sel = (lax.broadcasted_iota(jnp.int32, (128, 8), 0)
           == 16 * lax.broadcasted_iota(jnp.int32, (128, 8), 1)
           ).astype(jnp.float32)
    dn = (((0,), (1,)), ((), ()))
    om_ref[...] = lax.dot_general(sel, tm_ref[...], dn,
                                  preferred_element_type=jnp.float32)
    ol_ref[...] = lax.dot_general(sel, tl_ref[...], dn,
                                  preferred_element_type=jnp.float32)


_GBLK = _WROWS // 16


_compact = pl.pallas_call(
    _compact_body,
    grid=(16,),
    in_specs=[
        pl.BlockSpec((_GBLK, 128), lambda i: (i, 0)),
        pl.BlockSpec((_GBLK, 128), lambda i: (i, 0)),
    ],
    out_specs=[
        pl.BlockSpec((8, _GBLK), lambda i: (0, i)),
        pl.BlockSpec((8, _GBLK), lambda i: (0, i)),
    ],
    out_shape=[
        jax.ShapeDtypeStruct((8, _WROWS), jnp.float32),
        jax.ShapeDtypeStruct((8, _WROWS), jnp.float32),
    ],
)


def _finish_body(hm_ref, hl_ref, ptm_ref, ptl_ref, o_ref):
    hmk = jnp.sum(hm_ref[...], axis=0, keepdims=True)
    hlk = jnp.sum(hl_ref[...], axis=0, keepdims=True)
    acc = jnp.zeros((_F, 512), jnp.float32)
    for j in range(8):
        hmj = hmk[:, j * 512:(j + 1) * 512]
        hlj = hlk[:, j * 512:(j + 1) * 512]
        pmj = ptm_ref[j:j + 1, :].reshape(_F, 512)
        plj = ptl_ref[j:j + 1, :].reshape(_F, 512)
        acc = acc + pmj * hmj + plj * hlj
    o = jnp.sum(acc, axis=1, keepdims=True)
    o = o * (1.0 / _NPIX)
    o = jnp.clip(jnp.round(o * 2.0) * 0.5, -16.0, 15.5)
    o_ref[...] = o.T


_finish = pl.pallas_call(
    _finish_body,
    out_shape=jax.ShapeDtypeStruct((1, _F), jnp.float32),
)


def kernel(x_in, x_s, feature_msb, feature_lsb):
    tmf = feature_msb.transpose(1, 2, 3, 0).reshape(_WROWS, 128)
    tlf = feature_lsb.transpose(1, 2, 3, 0).reshape(_WROWS, 128)
    ptm, ptl = _compact(tmf, tlf)
    hm32, hl32 = _hist_sc(x_in, x_s)
    out = _finish(hm32, hl32, ptm, ptl)
    return out.reshape(1, _F, 1, 1)
